# baseline (speedup 1.0000x reference)
import jax
import jax.numpy as jnp
from jax import lax
from jax.experimental import pallas as pl
from jax.experimental.pallas import tpu as pltpu
from jax.experimental.pallas import tpu_sc as plsc

SEQ = 4096
D = 128
ENT_D = 64
VOCAB = 256
EEG_T = 512
THRESH = 0.5

N_CORES = 2
N_SUB = 16
N_W = N_CORES * N_SUB
PPT = SEQ // N_W
RPT = SEQ // N_SUB


def _prep_body(tb_ref, byte_emb_ref, byte_W_ref, byte_b_ref, ent_emb_ref,
               ent_W_ref, ent_b_ref, eeg_ref, eeg_W_ref, eeg_b_ref,
               fus_eeg_W_ref, fus_eeg_b_ref, out_W_ref, out_b_ref,
               seg_ref, ftab_ref, eegf_ref, eegp_ref, const_ref):
    h_tab = jnp.dot(byte_emb_ref[...], byte_W_ref[...],
                    preferred_element_type=jnp.float32) + byte_b_ref[...]
    ftab_ref[...] = jax.nn.gelu(h_tab)

    z = jnp.dot(ent_emb_ref[...], ent_W_ref[...],
                preferred_element_type=jnp.float32) + ent_b_ref[...]
    m = jnp.max(z, axis=1, keepdims=True)
    shifted = z - m
    ls = jnp.log(jnp.sum(jnp.exp(shifted), axis=1, keepdims=True))
    logp = shifted - ls
    p = jnp.exp(logp)
    ent_tab = -jnp.sum(p * logp, axis=1, keepdims=True)

    b_row = tb_ref[...]
    vv = lax.broadcasted_iota(jnp.int32, (VOCAB, SEQ), 0)
    oh = vv == b_row
    ent_pos = jnp.sum(jnp.where(oh, ent_tab, 0.0), axis=0, keepdims=True)

    pos = lax.broadcasted_iota(jnp.int32, (1, SEQ), 1)
    bnd = ((ent_pos > THRESH) & (pos > 0)).astype(jnp.int32)
    x = bnd
    k = 1
    while k < SEQ:
        shifted_x = jnp.concatenate(
            [jnp.zeros((1, k), jnp.int32), x[:, :SEQ - k]], axis=1)
        x = x + shifted_x
        k *= 2
    seg_ref[...] = x

    ef = jax.nn.gelu(jnp.dot(eeg_ref[...], eeg_W_ref[...],
                             preferred_element_type=jnp.float32)
                     + eeg_b_ref[...])
    eegf_ref[...] = ef
    ep_pre = jnp.dot(ef, fus_eeg_W_ref[...],
                     preferred_element_type=jnp.float32) + fus_eeg_b_ref[...]
    eeg_p = jnp.sum(ep_pre, axis=0, keepdims=True) / float(EEG_T)
    eegp_ref[...] = eeg_p
    const_ref[...] = jnp.dot(eeg_p, out_W_ref[D:, :],
                             preferred_element_type=jnp.float32) + out_b_ref[...]


def _prep_call(tb, byte_emb, byte_W, byte_b, ent_emb, ent_W, ent_b,
               eeg, eeg_W, eeg_b, fus_eeg_W, fus_eeg_b, out_W, out_b):
    return pl.pallas_call(
        _prep_body,
        out_shape=[
            jax.ShapeDtypeStruct((1, SEQ), jnp.int32),
            jax.ShapeDtypeStruct((VOCAB, D), jnp.float32),
            jax.ShapeDtypeStruct((EEG_T, D), jnp.float32),
            jax.ShapeDtypeStruct((1, D), jnp.float32),
            jax.ShapeDtypeStruct((1, 2 * D), jnp.float32),
        ],
    )(tb, byte_emb, byte_W, byte_b, ent_emb, ent_W, ent_b,
      eeg, eeg_W, eeg_b, fus_eeg_W, fus_eeg_b, out_W, out_b)


def _sc_body(bytes_hbm, seg_hbm, ftab_hbm, zeros_hbm, ones_hbm,
             sums_hbm, cnt_hbm,
             byt_v, seg_v, rows_v, ones_v, z_v, cnt_v,
             sh_acc, sem):
    cid = lax.axis_index("c")
    sid = lax.axis_index("s")
    wid = cid * N_SUB + sid
    base = wid * PPT
    rbase = sid * RPT
    obase = cid * SEQ + rbase

    l1 = pltpu.async_copy(bytes_hbm.at[pl.ds(base, PPT)], byt_v, sem)
    l2 = pltpu.async_copy(seg_hbm.at[0, pl.ds(base, PPT)], seg_v, sem)
    l3 = pltpu.async_copy(zeros_hbm, z_v, sem)
    l4 = pltpu.async_copy(ones_hbm, ones_v, sem)
    l1.wait()
    l2.wait()
    l3.wait()
    l4.wait()

    pltpu.sync_copy(z_v, sh_acc.at[pl.ds(rbase, PPT)])
    pltpu.sync_copy(z_v, sh_acc.at[pl.ds(rbase + PPT, PPT)])

    pltpu.async_copy(ftab_hbm.at[byt_v], rows_v, sem).wait()

    plsc.subcore_barrier()

    pltpu.sync_copy(rows_v, sh_acc.at[seg_v], add=True)

    plsc.subcore_barrier()

    for h in range(RPT // PPT):
        pltpu.sync_copy(sh_acc.at[pl.ds(rbase + h * PPT, PPT)], rows_v)
        pltpu.sync_copy(rows_v, sums_hbm.at[pl.ds(obase + h * PPT, PPT)])
    pltpu.sync_copy(z_v, sh_acc.at[pl.ds(rbase, PPT)])
    pltpu.sync_copy(z_v, sh_acc.at[pl.ds(rbase + PPT, PPT)])

    plsc.subcore_barrier()

    pltpu.sync_copy(ones_v, sh_acc.at[seg_v], add=True)

    plsc.subcore_barrier()

    for h in range(RPT // PPT):
        pltpu.sync_copy(sh_acc.at[pl.ds(rbase + h * PPT, PPT)], z_v)

        def _compress(r, _):
            cnt_v[h * PPT + r, pl.ds(0, 16)] = z_v[r, pl.ds(0, 16)]
            return 0

        lax.fori_loop(0, PPT, _compress, 0)
    pltpu.sync_copy(cnt_v, cnt_hbm.at[pl.ds(obase, RPT)])


def _sc_call(text_bytes, seg_ids, ftab):
    mesh = plsc.VectorSubcoreMesh(
        core_axis_name="c", subcore_axis_name="s")
    f = pl.kernel(
        _sc_body,
        out_type=[
            jax.ShapeDtypeStruct((N_CORES * SEQ, D), jnp.float32),
            jax.ShapeDtypeStruct((N_CORES * SEQ, 16), jnp.float32),
        ],
        mesh=mesh,
        scratch_types=[
            pltpu.VMEM((PPT,), jnp.int32),
            pltpu.VMEM((PPT,), jnp.int32),
            pltpu.VMEM((PPT, D), jnp.float32),
            pltpu.VMEM((PPT, D), jnp.float32),
            pltpu.VMEM((PPT, D), jnp.float32),
            pltpu.VMEM((RPT, 16), jnp.float32),
            pltpu.VMEM_SHARED((SEQ, D), jnp.float32),
            pltpu.SemaphoreType.DMA,
        ],
    )
    zeros = jnp.zeros((PPT, D), jnp.float32)
    ones = jnp.ones((PPT, D), jnp.float32)
    return f(text_bytes, seg_ids, ftab, zeros, ones)


def _fuse_body(sums0_ref, sums1_ref, cnt0_ref, cnt1_ref, fus_W_ref, fus_b_ref,
               outW_ref, const_ref, eegp_ref, logits_ref, tf_ref, joint_ref):
    sums = sums0_ref[...] + sums1_ref[...]
    cnt = cnt0_ref[:, 0:1] + cnt1_ref[:, 0:1]
    pooled = sums / cnt
    tf_ref[...] = pooled
    tp = jnp.dot(pooled, fus_W_ref[...],
                 preferred_element_type=jnp.float32) + fus_b_ref[...]
    joint_ref[...] = jnp.concatenate(
        [tp, jnp.broadcast_to(eegp_ref[...], tp.shape)], axis=1)
    logits_ref[...] = jnp.dot(tp, outW_ref[...],
                              preferred_element_type=jnp.float32) + const_ref[...]


def _fuse_call(sums, cnt, fus_W, fus_b, outW_top, const_row, eeg_p):
    nblk = 8
    blk = SEQ // nblk
    return pl.pallas_call(
        _fuse_body,
        grid=(nblk,),
        in_specs=[
            pl.BlockSpec((blk, D), lambda i: (i, 0)),
            pl.BlockSpec((blk, D), lambda i: (i + nblk, 0)),
            pl.BlockSpec((blk, 16), lambda i: (i, 0)),
            pl.BlockSpec((blk, 16), lambda i: (i + nblk, 0)),
            pl.BlockSpec((D, D), lambda i: (0, 0)),
            pl.BlockSpec((1, D), lambda i: (0, 0)),
            pl.BlockSpec((D, 2 * D), lambda i: (0, 0)),
            pl.BlockSpec((1, 2 * D), lambda i: (0, 0)),
            pl.BlockSpec((1, D), lambda i: (0, 0)),
        ],
        out_specs=[
            pl.BlockSpec((blk, 2 * D), lambda i: (i, 0)),
            pl.BlockSpec((blk, D), lambda i: (i, 0)),
            pl.BlockSpec((blk, 2 * D), lambda i: (i, 0)),
        ],
        out_shape=[
            jax.ShapeDtypeStruct((SEQ, 2 * D), jnp.float32),
            jax.ShapeDtypeStruct((SEQ, D), jnp.float32),
            jax.ShapeDtypeStruct((SEQ, 2 * D), jnp.float32),
        ],
    )(sums, sums, cnt, cnt, fus_W, fus_b, outW_top, const_row, eeg_p)


def kernel(text_bytes, eeg_data, byte_emb, byte_W, byte_b, ent_emb, ent_W,
           ent_b, eeg_W, eeg_b, fus_text_W, fus_text_b, fus_eeg_W, fus_eeg_b,
           out_W, out_b):
    tb = text_bytes.reshape(1, SEQ)
    seg2d, ftab, eegf, eeg_p, const_row = _prep_call(
        tb, byte_emb, byte_W, byte_b.reshape(1, D), ent_emb, ent_W,
        ent_b.reshape(1, VOCAB), eeg_data.reshape(EEG_T, ENT_D), eeg_W,
        eeg_b.reshape(1, D), fus_eeg_W, fus_eeg_b.reshape(1, D), out_W,
        out_b.reshape(1, 2 * D))

    sums, cnt = _sc_call(text_bytes, seg2d, ftab)

    logits, tf, joint = _fuse_call(
        sums, cnt, fus_text_W, fus_text_b.reshape(1, D), out_W,
        const_row, eeg_p)

    return (logits.reshape(1, SEQ, 2 * D), tf.reshape(1, SEQ, D),
            eegf.reshape(1, EEG_T, D), joint.reshape(1, SEQ, 2 * D))


# speedup vs baseline: 1.0756x; 1.0756x over previous
import jax
import jax.numpy as jnp
from jax import lax
from jax.experimental import pallas as pl
from jax.experimental.pallas import tpu as pltpu
from jax.experimental.pallas import tpu_sc as plsc

SEQ = 4096
D = 128
ENT_D = 64
VOCAB = 256
EEG_T = 512
THRESH = 0.5

N_CORES = 2
N_SUB = 16
N_W = N_CORES * N_SUB
PPT = SEQ // N_W
RPT = SEQ // N_SUB


def _prep_body(tb_ref, byte_emb_ref, byte_W_ref, byte_b_ref, ent_emb_ref,
               ent_W_ref, ent_b_ref, eeg_ref, eeg_W_ref, eeg_b_ref,
               fus_eeg_W_ref, fus_eeg_b_ref, out_W_ref, out_b_ref,
               seg_ref, ftab_ref, eegf_ref, eegp_ref, const_ref):
    h_tab = jnp.dot(byte_emb_ref[...], byte_W_ref[...],
                    preferred_element_type=jnp.float32) + byte_b_ref[...]
    ftab_ref[...] = jax.nn.gelu(h_tab)

    z = jnp.dot(ent_emb_ref[...], ent_W_ref[...],
                preferred_element_type=jnp.float32) + ent_b_ref[...]
    m = jnp.max(z, axis=1, keepdims=True)
    shifted = z - m
    ls = jnp.log(jnp.sum(jnp.exp(shifted), axis=1, keepdims=True))
    logp = shifted - ls
    p = jnp.exp(logp)
    ent_tab = -jnp.sum(p * logp, axis=1, keepdims=True)

    b_row = tb_ref[...]
    vv = lax.broadcasted_iota(jnp.int32, (VOCAB, SEQ), 0)
    oh = vv == b_row
    ent_pos = jnp.sum(jnp.where(oh, ent_tab, 0.0), axis=0, keepdims=True)

    pos = lax.broadcasted_iota(jnp.int32, (1, SEQ), 1)
    bnd = ((ent_pos > THRESH) & (pos > 0)).astype(jnp.int32)
    x = bnd
    k = 1
    while k < SEQ:
        shifted_x = jnp.concatenate(
            [jnp.zeros((1, k), jnp.int32), x[:, :SEQ - k]], axis=1)
        x = x + shifted_x
        k *= 2
    seg_ref[...] = x

    ef = jax.nn.gelu(jnp.dot(eeg_ref[...], eeg_W_ref[...],
                             preferred_element_type=jnp.float32)
                     + eeg_b_ref[...])
    eegf_ref[...] = ef
    ep_pre = jnp.dot(ef, fus_eeg_W_ref[...],
                     preferred_element_type=jnp.float32) + fus_eeg_b_ref[...]
    eeg_p = jnp.sum(ep_pre, axis=0, keepdims=True) / float(EEG_T)
    eegp_ref[...] = eeg_p
    const_ref[...] = jnp.dot(eeg_p, out_W_ref[D:, :],
                             preferred_element_type=jnp.float32) + out_b_ref[...]


def _prep_call(tb, byte_emb, byte_W, byte_b, ent_emb, ent_W, ent_b,
               eeg, eeg_W, eeg_b, fus_eeg_W, fus_eeg_b, out_W, out_b):
    return pl.pallas_call(
        _prep_body,
        out_shape=[
            jax.ShapeDtypeStruct((1, SEQ), jnp.int32),
            jax.ShapeDtypeStruct((VOCAB, D), jnp.float32),
            jax.ShapeDtypeStruct((EEG_T, D), jnp.float32),
            jax.ShapeDtypeStruct((1, D), jnp.float32),
            jax.ShapeDtypeStruct((1, 2 * D), jnp.float32),
        ],
    )(tb, byte_emb, byte_W, byte_b, ent_emb, ent_W, ent_b,
      eeg, eeg_W, eeg_b, fus_eeg_W, fus_eeg_b, out_W, out_b)


def _sc_body(bytes_hbm, seg_hbm, ftab_hbm, zeros_hbm, ones_hbm,
             sums_hbm, cnt_hbm,
             byt_a, byt_b, seg_a, seg_b, rows_v, ones_v, z_v, cnt_v,
             sh_acc, sem):
    cid = lax.axis_index("c")
    sid = lax.axis_index("s")
    base = sid * RPT

    l1 = pltpu.async_copy(bytes_hbm.at[pl.ds(base, PPT)], byt_a, sem)
    l2 = pltpu.async_copy(bytes_hbm.at[pl.ds(base + PPT, PPT)], byt_b, sem)
    l3 = pltpu.async_copy(seg_hbm.at[0, pl.ds(base, PPT)], seg_a, sem)
    l4 = pltpu.async_copy(seg_hbm.at[0, pl.ds(base + PPT, PPT)], seg_b, sem)
    l5 = pltpu.async_copy(zeros_hbm, z_v, sem)
    l6 = pltpu.async_copy(ones_hbm, ones_v, sem)
    l1.wait()
    l2.wait()
    l3.wait()
    l4.wait()
    l5.wait()
    l6.wait()

    pltpu.sync_copy(z_v, sh_acc.at[pl.ds(base, PPT)])
    pltpu.sync_copy(z_v, sh_acc.at[pl.ds(base + PPT, PPT)])

    @pl.when(cid == 0)
    def _():
        pltpu.async_copy(ftab_hbm.at[byt_a], rows_v, sem).wait()

    plsc.subcore_barrier()

    @pl.when(cid == 0)
    def _():
        pltpu.sync_copy(rows_v, sh_acc.at[seg_a], add=True)
        pltpu.async_copy(ftab_hbm.at[byt_b], rows_v, sem).wait()
        pltpu.sync_copy(rows_v, sh_acc.at[seg_b], add=True)

    @pl.when(cid == 1)
    def _():
        pltpu.sync_copy(ones_v, sh_acc.at[seg_a], add=True)
        pltpu.sync_copy(ones_v, sh_acc.at[seg_b], add=True)

    plsc.subcore_barrier()

    @pl.when(cid == 0)
    def _():
        for h in range(RPT // PPT):
            pltpu.sync_copy(sh_acc.at[pl.ds(base + h * PPT, PPT)], rows_v)
            pltpu.sync_copy(rows_v, sums_hbm.at[pl.ds(base + h * PPT, PPT)])

    @pl.when(cid == 1)
    def _():
        for h in range(RPT // PPT):
            pltpu.sync_copy(sh_acc.at[pl.ds(base + h * PPT, PPT)], z_v)

            def _compress(r, _):
                cnt_v[h * PPT + r, pl.ds(0, 16)] = z_v[r, pl.ds(0, 16)]
                return 0

            lax.fori_loop(0, PPT, _compress, 0)
        pltpu.sync_copy(cnt_v, cnt_hbm.at[pl.ds(base, RPT)])


def _sc_call(text_bytes, seg_ids, ftab):
    mesh = plsc.VectorSubcoreMesh(
        core_axis_name="c", subcore_axis_name="s")
    f = pl.kernel(
        _sc_body,
        out_type=[
            jax.ShapeDtypeStruct((SEQ, D), jnp.float32),
            jax.ShapeDtypeStruct((SEQ, 16), jnp.float32),
        ],
        mesh=mesh,
        scratch_types=[
            pltpu.VMEM((PPT,), jnp.int32),
            pltpu.VMEM((PPT,), jnp.int32),
            pltpu.VMEM((PPT,), jnp.int32),
            pltpu.VMEM((PPT,), jnp.int32),
            pltpu.VMEM((PPT, D), jnp.float32),
            pltpu.VMEM((PPT, D), jnp.float32),
            pltpu.VMEM((PPT, D), jnp.float32),
            pltpu.VMEM((RPT, 16), jnp.float32),
            pltpu.VMEM_SHARED((SEQ, D), jnp.float32),
            pltpu.SemaphoreType.DMA,
        ],
    )
    zeros = jnp.zeros((PPT, D), jnp.float32)
    ones = jnp.ones((PPT, D), jnp.float32)
    return f(text_bytes, seg_ids, ftab, zeros, ones)


def _fuse_body(sums_ref, cnt_ref, fus_W_ref, fus_b_ref,
               outW_ref, const_ref, eegp_ref, logits_ref, tf_ref, joint_ref):
    pooled = sums_ref[...] / cnt_ref[:, 0:1]
    tf_ref[...] = pooled
    tp = jnp.dot(pooled, fus_W_ref[...],
                 preferred_element_type=jnp.float32) + fus_b_ref[...]
    joint_ref[...] = jnp.concatenate(
        [tp, jnp.broadcast_to(eegp_ref[...], tp.shape)], axis=1)
    logits_ref[...] = jnp.dot(tp, outW_ref[...],
                              preferred_element_type=jnp.float32) + const_ref[...]


def _fuse_call(sums, cnt, fus_W, fus_b, outW_top, const_row, eeg_p):
    nblk = 8
    blk = SEQ // nblk
    return pl.pallas_call(
        _fuse_body,
        grid=(nblk,),
        in_specs=[
            pl.BlockSpec((blk, D), lambda i: (i, 0)),
            pl.BlockSpec((blk, 16), lambda i: (i, 0)),
            pl.BlockSpec((D, D), lambda i: (0, 0)),
            pl.BlockSpec((1, D), lambda i: (0, 0)),
            pl.BlockSpec((D, 2 * D), lambda i: (0, 0)),
            pl.BlockSpec((1, 2 * D), lambda i: (0, 0)),
            pl.BlockSpec((1, D), lambda i: (0, 0)),
        ],
        out_specs=[
            pl.BlockSpec((blk, 2 * D), lambda i: (i, 0)),
            pl.BlockSpec((blk, D), lambda i: (i, 0)),
            pl.BlockSpec((blk, 2 * D), lambda i: (i, 0)),
        ],
        out_shape=[
            jax.ShapeDtypeStruct((SEQ, 2 * D), jnp.float32),
            jax.ShapeDtypeStruct((SEQ, D), jnp.float32),
            jax.ShapeDtypeStruct((SEQ, 2 * D), jnp.float32),
        ],
    )(sums, cnt, fus_W, fus_b, outW_top, const_row, eeg_p)


def kernel(text_bytes, eeg_data, byte_emb, byte_W, byte_b, ent_emb, ent_W,
           ent_b, eeg_W, eeg_b, fus_text_W, fus_text_b, fus_eeg_W, fus_eeg_b,
           out_W, out_b):
    tb = text_bytes.reshape(1, SEQ)
    seg2d, ftab, eegf, eeg_p, const_row = _prep_call(
        tb, byte_emb, byte_W, byte_b.reshape(1, D), ent_emb, ent_W,
        ent_b.reshape(1, VOCAB), eeg_data.reshape(EEG_T, ENT_D), eeg_W,
        eeg_b.reshape(1, D), fus_eeg_W, fus_eeg_b.reshape(1, D), out_W,
        out_b.reshape(1, 2 * D))

    sums, cnt = _sc_call(text_bytes, seg2d, ftab)

    logits, tf, joint = _fuse_call(
        sums, cnt, fus_text_W, fus_text_b.reshape(1, D), out_W,
        const_row, eeg_p)

    return (logits.reshape(1, SEQ, 2 * D), tf.reshape(1, SEQ, D),
            eegf.reshape(1, EEG_T, D), joint.reshape(1, SEQ, 2 * D))


# speedup vs baseline: 1.1652x; 1.0833x over previous
import jax
import jax.numpy as jnp
from jax import lax
from jax.experimental import pallas as pl
from jax.experimental.pallas import tpu as pltpu
from jax.experimental.pallas import tpu_sc as plsc

SEQ = 4096
D = 128
ENT_D = 64
VOCAB = 256
EEG_T = 512
THRESH = 0.5

N_CORES = 2
N_SUB = 16
N_W = N_CORES * N_SUB
PPT = SEQ // N_W
RPT = SEQ // N_SUB


def _prep_body(tb_ref, byte_emb_ref, byte_W_ref, byte_b_ref, ent_emb_ref,
               ent_W_ref, ent_b_ref, eeg_ref, eeg_W_ref, eeg_b_ref,
               fus_eeg_W_ref, fus_eeg_b_ref, out_W_ref, out_b_ref,
               seg_ref, ftab_ref, eegf_ref, eegp_ref, const_ref):
    h_tab = jnp.dot(byte_emb_ref[...], byte_W_ref[...],
                    preferred_element_type=jnp.float32) + byte_b_ref[...]
    ftab_ref[...] = jax.nn.gelu(h_tab)

    z = jnp.dot(ent_emb_ref[...], ent_W_ref[...],
                preferred_element_type=jnp.float32) + ent_b_ref[...]
    m = jnp.max(z, axis=1, keepdims=True)
    shifted = z - m
    ls = jnp.log(jnp.sum(jnp.exp(shifted), axis=1, keepdims=True))
    logp = shifted - ls
    p = jnp.exp(logp)
    ent_tab = -jnp.sum(p * logp, axis=1, keepdims=True)

    b_row = tb_ref[...]
    vv = lax.broadcasted_iota(jnp.int32, (VOCAB, SEQ), 0)
    oh = vv == b_row
    ent_pos = jnp.sum(jnp.where(oh, ent_tab, 0.0), axis=0, keepdims=True)

    pos = lax.broadcasted_iota(jnp.int32, (1, SEQ), 1)
    bnd = ((ent_pos > THRESH) & (pos > 0)).astype(jnp.int32)
    x = bnd
    k = 1
    while k < SEQ:
        shifted_x = jnp.concatenate(
            [jnp.zeros((1, k), jnp.int32), x[:, :SEQ - k]], axis=1)
        x = x + shifted_x
        k *= 2
    seg_ref[...] = x

    ef = jax.nn.gelu(jnp.dot(eeg_ref[...], eeg_W_ref[...],
                             preferred_element_type=jnp.float32)
                     + eeg_b_ref[...])
    eegf_ref[...] = ef
    ep_pre = jnp.dot(ef, fus_eeg_W_ref[...],
                     preferred_element_type=jnp.float32) + fus_eeg_b_ref[...]
    eeg_p = jnp.sum(ep_pre, axis=0, keepdims=True) / float(EEG_T)
    eegp_ref[...] = eeg_p
    const_ref[...] = jnp.dot(eeg_p, out_W_ref[D:, :],
                             preferred_element_type=jnp.float32) + out_b_ref[...]


def _prep_call(tb, byte_emb, byte_W, byte_b, ent_emb, ent_W, ent_b,
               eeg, eeg_W, eeg_b, fus_eeg_W, fus_eeg_b, out_W, out_b):
    return pl.pallas_call(
        _prep_body,
        out_shape=[
            jax.ShapeDtypeStruct((1, SEQ), jnp.int32),
            jax.ShapeDtypeStruct((VOCAB, D), jnp.float32),
            jax.ShapeDtypeStruct((EEG_T, D), jnp.float32),
            jax.ShapeDtypeStruct((1, D), jnp.float32),
            jax.ShapeDtypeStruct((1, 2 * D), jnp.float32),
        ],
    )(tb, byte_emb, byte_W, byte_b, ent_emb, ent_W, ent_b,
      eeg, eeg_W, eeg_b, fus_eeg_W, fus_eeg_b, out_W, out_b)


def _sc_body(bytes_hbm, seg_hbm, ftab_hbm, zeros_hbm, ones_hbm,
             sums_hbm, cnt_hbm,
             byt_a, byt_b, seg_a, seg_b, rows_v, ones_v, z_v, cnt_v,
             sh_acc, sem):
    cid = lax.axis_index("c")
    sid = lax.axis_index("s")
    base = sid * RPT

    l1 = pltpu.async_copy(bytes_hbm.at[pl.ds(base, PPT)], byt_a, sem)
    l2 = pltpu.async_copy(bytes_hbm.at[pl.ds(base + PPT, PPT)], byt_b, sem)
    l3 = pltpu.async_copy(seg_hbm.at[0, pl.ds(base, PPT)], seg_a, sem)
    l4 = pltpu.async_copy(seg_hbm.at[0, pl.ds(base + PPT, PPT)], seg_b, sem)
    l5 = pltpu.async_copy(zeros_hbm, z_v, sem)
    l6 = pltpu.async_copy(ones_hbm, ones_v, sem)
    l1.wait()
    l2.wait()
    l3.wait()
    l4.wait()
    l5.wait()
    l6.wait()

    pltpu.sync_copy(z_v, sh_acc.at[pl.ds(base, PPT)])
    pltpu.sync_copy(z_v, sh_acc.at[pl.ds(base + PPT, PPT)])

    @pl.when(cid == 0)
    def _():
        pltpu.async_copy(ftab_hbm.at[byt_a], rows_v, sem).wait()

    plsc.subcore_barrier()

    @pl.when(cid == 0)
    def _():
        pltpu.sync_copy(rows_v, sh_acc.at[seg_a], add=True)
        pltpu.async_copy(ftab_hbm.at[byt_b], rows_v, sem).wait()
        pltpu.sync_copy(rows_v, sh_acc.at[seg_b], add=True)

    @pl.when(cid == 1)
    def _():
        pltpu.sync_copy(ones_v, sh_acc.at[seg_a], add=True)
        pltpu.sync_copy(ones_v, sh_acc.at[seg_b], add=True)

    plsc.subcore_barrier()

    @pl.when(cid == 0)
    def _():
        for h in range(RPT // PPT):
            pltpu.sync_copy(sh_acc.at[pl.ds(base + h * PPT, PPT)], rows_v)
            pltpu.sync_copy(rows_v, sums_hbm.at[pl.ds(base + h * PPT, PPT)])

    @pl.when(cid == 1)
    def _():
        for h in range(RPT // PPT):
            pltpu.sync_copy(sh_acc.at[pl.ds(base + h * PPT, PPT)], z_v)

            def _compress(r, _):
                cnt_v[h * PPT + r, pl.ds(0, 16)] = z_v[r, pl.ds(0, 16)]
                return 0

            lax.fori_loop(0, PPT, _compress, 0)
        pltpu.sync_copy(cnt_v, cnt_hbm.at[pl.ds(base, RPT)])


def _sc_call(text_bytes, seg_ids, ftab):
    mesh = plsc.VectorSubcoreMesh(
        core_axis_name="c", subcore_axis_name="s")
    f = pl.kernel(
        _sc_body,
        out_type=[
            jax.ShapeDtypeStruct((SEQ, D), jnp.float32),
            jax.ShapeDtypeStruct((SEQ, 16), jnp.float32),
        ],
        mesh=mesh,
        scratch_types=[
            pltpu.VMEM((PPT,), jnp.int32),
            pltpu.VMEM((PPT,), jnp.int32),
            pltpu.VMEM((PPT,), jnp.int32),
            pltpu.VMEM((PPT,), jnp.int32),
            pltpu.VMEM((PPT, D), jnp.float32),
            pltpu.VMEM((PPT, D), jnp.float32),
            pltpu.VMEM((PPT, D), jnp.float32),
            pltpu.VMEM((RPT, 16), jnp.float32),
            pltpu.VMEM_SHARED((SEQ, D), jnp.float32),
            pltpu.SemaphoreType.DMA,
        ],
    )
    zeros = jnp.zeros((PPT, D), jnp.float32)
    ones = jnp.ones((PPT, D), jnp.float32)
    return f(text_bytes, seg_ids, ftab, zeros, ones)


def _fuse_body(sums_ref, cnt_ref, fus_W_ref, fus_b_ref,
               outW_ref, const_ref, eegp_ref, logits_ref, tf_ref, joint_ref):
    pooled = sums_ref[...] / cnt_ref[:, 0:1]
    tf_ref[...] = pooled
    tp = jnp.dot(pooled, fus_W_ref[...],
                 preferred_element_type=jnp.float32) + fus_b_ref[...]
    joint_ref[...] = jnp.concatenate(
        [tp, jnp.broadcast_to(eegp_ref[...], tp.shape)], axis=1)
    logits_ref[...] = jnp.dot(tp, outW_ref[...],
                              preferred_element_type=jnp.float32) + const_ref[...]


def _fuse_call(sums, cnt, fus_W, fus_b, outW_top, const_row, eeg_p):
    nblk = 4
    blk = SEQ // nblk
    return pl.pallas_call(
        _fuse_body,
        grid=(nblk,),
        in_specs=[
            pl.BlockSpec((blk, D), lambda i: (i, 0)),
            pl.BlockSpec((blk, 16), lambda i: (i, 0)),
            pl.BlockSpec((D, D), lambda i: (0, 0)),
            pl.BlockSpec((1, D), lambda i: (0, 0)),
            pl.BlockSpec((D, 2 * D), lambda i: (0, 0)),
            pl.BlockSpec((1, 2 * D), lambda i: (0, 0)),
            pl.BlockSpec((1, D), lambda i: (0, 0)),
        ],
        out_specs=[
            pl.BlockSpec((blk, 2 * D), lambda i: (i, 0)),
            pl.BlockSpec((blk, D), lambda i: (i, 0)),
            pl.BlockSpec((blk, 2 * D), lambda i: (i, 0)),
        ],
        out_shape=[
            jax.ShapeDtypeStruct((SEQ, 2 * D), jnp.float32),
            jax.ShapeDtypeStruct((SEQ, D), jnp.float32),
            jax.ShapeDtypeStruct((SEQ, 2 * D), jnp.float32),
        ],
    )(sums, cnt, fus_W, fus_b, outW_top, const_row, eeg_p)


def kernel(text_bytes, eeg_data, byte_emb, byte_W, byte_b, ent_emb, ent_W,
           ent_b, eeg_W, eeg_b, fus_text_W, fus_text_b, fus_eeg_W, fus_eeg_b,
           out_W, out_b):
    tb = text_bytes.reshape(1, SEQ)
    seg2d, ftab, eegf, eeg_p, const_row = _prep_call(
        tb, byte_emb, byte_W, byte_b.reshape(1, D), ent_emb, ent_W,
        ent_b.reshape(1, VOCAB), eeg_data.reshape(EEG_T, ENT_D), eeg_W,
        eeg_b.reshape(1, D), fus_eeg_W, fus_eeg_b.reshape(1, D), out_W,
        out_b.reshape(1, 2 * D))

    sums, cnt = _sc_call(text_bytes, seg2d, ftab)

    logits, tf, joint = _fuse_call(
        sums, cnt, fus_text_W, fus_text_b.reshape(1, D), out_W,
        const_row, eeg_p)

    return (logits.reshape(1, SEQ, 2 * D), tf.reshape(1, SEQ, D),
            eegf.reshape(1, EEG_T, D), joint.reshape(1, SEQ, 2 * D))


# speedup vs baseline: 1.1784x; 1.0113x over previous
import jax
import jax.numpy as jnp
from jax import lax
from jax.experimental import pallas as pl
from jax.experimental.pallas import tpu as pltpu
from jax.experimental.pallas import tpu_sc as plsc

SEQ = 4096
D = 128
ENT_D = 64
VOCAB = 256
EEG_T = 512
THRESH = 0.5

N_CORES = 2
N_SUB = 16
N_W = N_CORES * N_SUB
PPT = SEQ // N_W
RPT = SEQ // N_SUB


def _prep_body(tb_ref, byte_emb_ref, byte_W_ref, byte_b_ref, ent_emb_ref,
               ent_W_ref, ent_b_ref, eeg_ref, eeg_W_ref, eeg_b_ref,
               fus_eeg_W_ref, fus_eeg_b_ref, out_W_ref, out_b_ref,
               seg_ref, ftab_ref, eegf_ref, eegp_ref, const_ref):
    h_tab = jnp.dot(byte_emb_ref[...], byte_W_ref[...],
                    preferred_element_type=jnp.float32) + byte_b_ref[...]
    ftab_ref[...] = jax.nn.gelu(h_tab)

    z = jnp.dot(ent_emb_ref[...], ent_W_ref[...],
                preferred_element_type=jnp.float32) + ent_b_ref[...]
    m = jnp.max(z, axis=1, keepdims=True)
    shifted = z - m
    ls = jnp.log(jnp.sum(jnp.exp(shifted), axis=1, keepdims=True))
    logp = shifted - ls
    p = jnp.exp(logp)
    ent_tab = -jnp.sum(p * logp, axis=1, keepdims=True)

    b_row = tb_ref[...]
    vv = lax.broadcasted_iota(jnp.int32, (VOCAB, SEQ), 0)
    oh = vv == b_row
    ent_pos = jnp.sum(jnp.where(oh, ent_tab, 0.0), axis=0, keepdims=True)

    pos = lax.broadcasted_iota(jnp.int32, (1, SEQ), 1)
    bnd = ((ent_pos > THRESH) & (pos > 0)).astype(jnp.int32)
    x = bnd
    k = 1
    while k < SEQ:
        shifted_x = jnp.concatenate(
            [jnp.zeros((1, k), jnp.int32), x[:, :SEQ - k]], axis=1)
        x = x + shifted_x
        k *= 2
    seg_ref[...] = x

    ef = jax.nn.gelu(jnp.dot(eeg_ref[...], eeg_W_ref[...],
                             preferred_element_type=jnp.float32)
                     + eeg_b_ref[...])
    eegf_ref[...] = ef
    ep_pre = jnp.dot(ef, fus_eeg_W_ref[...],
                     preferred_element_type=jnp.float32) + fus_eeg_b_ref[...]
    eeg_p = jnp.sum(ep_pre, axis=0, keepdims=True) / float(EEG_T)
    eegp_ref[...] = eeg_p
    const_ref[...] = jnp.dot(eeg_p, out_W_ref[D:, :],
                             preferred_element_type=jnp.float32) + out_b_ref[...]


def _prep_call(tb, byte_emb, byte_W, byte_b, ent_emb, ent_W, ent_b,
               eeg, eeg_W, eeg_b, fus_eeg_W, fus_eeg_b, out_W, out_b):
    return pl.pallas_call(
        _prep_body,
        out_shape=[
            jax.ShapeDtypeStruct((1, SEQ), jnp.int32),
            jax.ShapeDtypeStruct((VOCAB, D), jnp.float32),
            jax.ShapeDtypeStruct((EEG_T, D), jnp.float32),
            jax.ShapeDtypeStruct((1, D), jnp.float32),
            jax.ShapeDtypeStruct((1, 2 * D), jnp.float32),
        ],
    )(tb, byte_emb, byte_W, byte_b, ent_emb, ent_W, ent_b,
      eeg, eeg_W, eeg_b, fus_eeg_W, fus_eeg_b, out_W, out_b)


def _sc_body(bytes_hbm, seg_hbm, ftab_hbm, zeros_hbm, ones_hbm,
             sums_hbm, cnt_hbm,
             byt_a, byt_b, seg_a, seg_b, rows_v, ones_v, z_v, cnt_v,
             sh_acc, sem):
    cid = lax.axis_index("c")
    sid = lax.axis_index("s")
    base = sid * RPT

    l1 = pltpu.async_copy(bytes_hbm.at[pl.ds(base, PPT)], byt_a, sem)
    l2 = pltpu.async_copy(bytes_hbm.at[pl.ds(base + PPT, PPT)], byt_b, sem)
    l3 = pltpu.async_copy(seg_hbm.at[0, pl.ds(base, PPT)], seg_a, sem)
    l4 = pltpu.async_copy(seg_hbm.at[0, pl.ds(base + PPT, PPT)], seg_b, sem)
    l5 = pltpu.async_copy(zeros_hbm, z_v, sem)
    l6 = pltpu.async_copy(ones_hbm, ones_v, sem)
    l1.wait()
    l2.wait()
    l3.wait()
    l4.wait()
    l5.wait()
    l6.wait()

    pltpu.sync_copy(z_v, sh_acc.at[pl.ds(base, PPT)])
    pltpu.sync_copy(z_v, sh_acc.at[pl.ds(base + PPT, PPT)])

    @pl.when(cid == 0)
    def _():
        pltpu.async_copy(ftab_hbm.at[byt_a], rows_v, sem).wait()

    plsc.subcore_barrier()

    @pl.when(cid == 0)
    def _():
        pltpu.sync_copy(rows_v, sh_acc.at[seg_a], add=True)
        pltpu.async_copy(ftab_hbm.at[byt_b], rows_v, sem).wait()
        pltpu.sync_copy(rows_v, sh_acc.at[seg_b], add=True)

    @pl.when(cid == 1)
    def _():
        pltpu.sync_copy(ones_v, sh_acc.at[seg_a], add=True)
        pltpu.sync_copy(ones_v, sh_acc.at[seg_b], add=True)

    plsc.subcore_barrier()

    @pl.when(cid == 0)
    def _():
        for h in range(RPT // PPT):
            pltpu.sync_copy(sh_acc.at[pl.ds(base + h * PPT, PPT)], rows_v)
            pltpu.sync_copy(rows_v, sums_hbm.at[pl.ds(base + h * PPT, PPT)])

    @pl.when(cid == 1)
    def _():
        for h in range(RPT // PPT):
            pltpu.sync_copy(sh_acc.at[pl.ds(base + h * PPT, PPT)], z_v)

            def _compress(r, _):
                cnt_v[h * PPT + r, pl.ds(0, 16)] = z_v[r, pl.ds(0, 16)]
                return 0

            lax.fori_loop(0, PPT, _compress, 0)
        pltpu.sync_copy(cnt_v, cnt_hbm.at[pl.ds(base, RPT)])


def _sc_call(text_bytes, seg_ids, ftab):
    mesh = plsc.VectorSubcoreMesh(
        core_axis_name="c", subcore_axis_name="s")
    f = pl.kernel(
        _sc_body,
        out_type=[
            jax.ShapeDtypeStruct((SEQ, D), jnp.float32),
            jax.ShapeDtypeStruct((SEQ, 16), jnp.float32),
        ],
        mesh=mesh,
        scratch_types=[
            pltpu.VMEM((PPT,), jnp.int32),
            pltpu.VMEM((PPT,), jnp.int32),
            pltpu.VMEM((PPT,), jnp.int32),
            pltpu.VMEM((PPT,), jnp.int32),
            pltpu.VMEM((PPT, D), jnp.float32),
            pltpu.VMEM((PPT, D), jnp.float32),
            pltpu.VMEM((PPT, D), jnp.float32),
            pltpu.VMEM((RPT, 16), jnp.float32),
            pltpu.VMEM_SHARED((SEQ, D), jnp.float32),
            pltpu.SemaphoreType.DMA,
        ],
    )
    zeros = jnp.zeros((PPT, D), jnp.float32)
    ones = jnp.ones((PPT, D), jnp.float32)
    return f(text_bytes, seg_ids, ftab, zeros, ones)


def _fuse_body(sums_ref, cnt_ref, fus_W_ref, fus_b_ref,
               outW_ref, const_ref, eegp_ref, logits_ref, tf_ref, joint_ref):
    pooled = sums_ref[...] / cnt_ref[:, 0:1]
    tf_ref[...] = pooled
    tp = jnp.dot(pooled, fus_W_ref[...],
                 preferred_element_type=jnp.float32) + fus_b_ref[...]
    joint_ref[...] = jnp.concatenate(
        [tp, jnp.broadcast_to(eegp_ref[...], tp.shape)], axis=1)
    logits_ref[...] = jnp.dot(tp, outW_ref[...],
                              preferred_element_type=jnp.float32) + const_ref[...]


def _fuse_call(sums, cnt, fus_W, fus_b, outW_top, const_row, eeg_p):
    nblk = 2
    blk = SEQ // nblk
    return pl.pallas_call(
        _fuse_body,
        grid=(nblk,),
        in_specs=[
            pl.BlockSpec((blk, D), lambda i: (i, 0)),
            pl.BlockSpec((blk, 16), lambda i: (i, 0)),
            pl.BlockSpec((D, D), lambda i: (0, 0)),
            pl.BlockSpec((1, D), lambda i: (0, 0)),
            pl.BlockSpec((D, 2 * D), lambda i: (0, 0)),
            pl.BlockSpec((1, 2 * D), lambda i: (0, 0)),
            pl.BlockSpec((1, D), lambda i: (0, 0)),
        ],
        out_specs=[
            pl.BlockSpec((blk, 2 * D), lambda i: (i, 0)),
            pl.BlockSpec((blk, D), lambda i: (i, 0)),
            pl.BlockSpec((blk, 2 * D), lambda i: (i, 0)),
        ],
        out_shape=[
            jax.ShapeDtypeStruct((SEQ, 2 * D), jnp.float32),
            jax.ShapeDtypeStruct((SEQ, D), jnp.float32),
            jax.ShapeDtypeStruct((SEQ, 2 * D), jnp.float32),
        ],
    )(sums, cnt, fus_W, fus_b, outW_top, const_row, eeg_p)


def kernel(text_bytes, eeg_data, byte_emb, byte_W, byte_b, ent_emb, ent_W,
           ent_b, eeg_W, eeg_b, fus_text_W, fus_text_b, fus_eeg_W, fus_eeg_b,
           out_W, out_b):
    tb = text_bytes.reshape(1, SEQ)
    seg2d, ftab, eegf, eeg_p, const_row = _prep_call(
        tb, byte_emb, byte_W, byte_b.reshape(1, D), ent_emb, ent_W,
        ent_b.reshape(1, VOCAB), eeg_data.reshape(EEG_T, ENT_D), eeg_W,
        eeg_b.reshape(1, D), fus_eeg_W, fus_eeg_b.reshape(1, D), out_W,
        out_b.reshape(1, 2 * D))

    sums, cnt = _sc_call(text_bytes, seg2d, ftab)

    logits, tf, joint = _fuse_call(
        sums, cnt, fus_text_W, fus_text_b.reshape(1, D), out_W,
        const_row, eeg_p)

    return (logits.reshape(1, SEQ, 2 * D), tf.reshape(1, SEQ, D),
            eegf.reshape(1, EEG_T, D), joint.reshape(1, SEQ, 2 * D))
